# Initial kernel scaffold; baseline (speedup 1.0000x reference)
import jax
import jax.numpy as jnp
from jax.experimental import pallas as pl


def kernel(x, edge_index, W1, b1, W2, b2, W3, b3, W4, b4, gamma1, beta1, gamma2, beta2, gamma3, beta3):
    raise NotImplementedError("write your pallas kernel here")


# speedup vs baseline: 6.9309x; 6.9309x over previous
import functools
import math

import jax
import jax.numpy as jnp
from jax import lax
from jax.experimental import pallas as pl
from jax.experimental.pallas import tpu as pltpu
from jax.experimental.pallas import tpu_sc as plsc

N = 10000
E = 320000
D = 128
BN_EPS = 1e-5

NC = 2
NS = 16
NW = NC * NS

N_PAD = 10240
CHUNK = 128
NCHUNK = 79
EPT = CHUNK * NCHUNK
E_PAD = EPT * NW
ROWS_PER_TILE = N_PAD // NS

BM = 1280
GRID = N_PAD // BM


def _make_propagate():
  mesh = plsc.VectorSubcoreMesh(core_axis_name="c", subcore_axis_name="s",
                                num_cores=NC, num_subcores=NS)

  @functools.partial(
      pl.kernel,
      out_type=jax.ShapeDtypeStruct((NC, N_PAD, D), jnp.float32),
      mesh=mesh,
      scratch_types=[
          pltpu.VMEM((CHUNK,), jnp.int32),
          pltpu.VMEM((CHUNK,), jnp.int32),
          pltpu.VMEM((CHUNK, D), jnp.float32),
          pltpu.VMEM_SHARED((N_PAD, D), jnp.float32),
          pltpu.SemaphoreType.DMA,
      ],
  )
  def prop(g_hbm, src_hbm, dst_hbm, zeros_hbm, out_hbm,
           sidx, didx, rows, acc, sem):
    cid = lax.axis_index("c")
    sid = lax.axis_index("s")
    tid = cid * NS + sid

    pltpu.sync_copy(zeros_hbm, acc.at[pl.ds(sid * ROWS_PER_TILE, ROWS_PER_TILE)])
    plsc.subcore_barrier()

    base = tid * EPT

    def body(i, carry):
      off = base + i * CHUNK
      pltpu.sync_copy(src_hbm.at[pl.ds(off, CHUNK)], sidx)
      pltpu.sync_copy(dst_hbm.at[pl.ds(off, CHUNK)], didx)
      pltpu.async_copy(g_hbm.at[sidx], rows, sem).wait()
      pltpu.sync_copy(rows, acc.at[didx], add=True)
      return carry

    lax.fori_loop(0, NCHUNK, body, 0)
    plsc.subcore_barrier()

    pltpu.sync_copy(acc.at[pl.ds(sid * ROWS_PER_TILE, ROWS_PER_TILE)],
                    out_hbm.at[cid, pl.ds(sid * ROWS_PER_TILE, ROWS_PER_TILE)])

  return prop


@functools.cache
def _get_propagate():
  return _make_propagate()


def _propagate(g, srcp, dstp, zeros_t):
  return _get_propagate()(g, srcp, dstp, zeros_t)


def _prep_dinv(deg_partials):
  def body(p_ref, o_ref):
    i = pl.program_id(0)
    deg = 1.0 + p_ref[0] + p_ref[1]
    dinv = lax.rsqrt(deg)
    row = i * BM + lax.broadcasted_iota(jnp.int32, (BM, D), 0)
    o_ref[...] = jnp.where(row < N, dinv, 0.0)

  return pl.pallas_call(
      body,
      grid=(GRID,),
      in_specs=[pl.BlockSpec((NC, BM, D), lambda i: (0, i, 0))],
      out_specs=pl.BlockSpec((BM, D), lambda i: (i, 0)),
      out_shape=jax.ShapeDtypeStruct((N_PAD, D), jnp.float32),
  )(deg_partials)


def _matmul_scale(x, W, dinv_b):
  def body(x_ref, w_ref, d_ref, o_ref):
    o_ref[...] = d_ref[...] * jnp.dot(x_ref[...], w_ref[...],
                                      preferred_element_type=jnp.float32)

  return pl.pallas_call(
      body,
      grid=(GRID,),
      in_specs=[
          pl.BlockSpec((BM, D), lambda i: (i, 0)),
          pl.BlockSpec((D, D), lambda i: (0, 0)),
          pl.BlockSpec((BM, D), lambda i: (i, 0)),
      ],
      out_specs=pl.BlockSpec((BM, D), lambda i: (i, 0)),
      out_shape=jax.ShapeDtypeStruct((N_PAD, D), jnp.float32),
  )(x, W, dinv_b)


def _combine_matmul(P, g, dinv_b, bvec, gamma, beta, W):
  bn_c = float(1.0 / math.sqrt(1.0 + BN_EPS))

  def body(p_ref, g_ref, d_ref, b_ref, ga_ref, be_ref, w_ref, o_ref):
    z = d_ref[...] * (p_ref[0] + p_ref[1] + g_ref[...]) + b_ref[...]
    z = jnp.maximum(z, 0.0) * (ga_ref[...] * bn_c) + be_ref[...]
    o_ref[...] = d_ref[...] * jnp.dot(z, w_ref[...],
                                      preferred_element_type=jnp.float32)

  return pl.pallas_call(
      body,
      grid=(GRID,),
      in_specs=[
          pl.BlockSpec((NC, BM, D), lambda i: (0, i, 0)),
          pl.BlockSpec((BM, D), lambda i: (i, 0)),
          pl.BlockSpec((BM, D), lambda i: (i, 0)),
          pl.BlockSpec((1, D), lambda i: (0, 0)),
          pl.BlockSpec((1, D), lambda i: (0, 0)),
          pl.BlockSpec((1, D), lambda i: (0, 0)),
          pl.BlockSpec((D, D), lambda i: (0, 0)),
      ],
      out_specs=pl.BlockSpec((BM, D), lambda i: (i, 0)),
      out_shape=jax.ShapeDtypeStruct((N_PAD, D), jnp.float32),
  )(P, g, dinv_b, bvec, gamma, beta, W)


def _final_mean(P, g, dinv_b, bvec):
  def body(p_ref, g_ref, d_ref, b_ref, o_ref):
    i = pl.program_id(0)
    z = d_ref[...] * (p_ref[0] + p_ref[1] + g_ref[...])
    row = i * BM + lax.broadcasted_iota(jnp.int32, (BM, D), 0)
    z = jnp.where(row < N, z, 0.0)
    part = jnp.sum(z, axis=0, keepdims=True)

    @pl.when(i == 0)
    def _():
      o_ref[...] = jnp.zeros_like(o_ref)

    o_ref[...] += part

    @pl.when(i == GRID - 1)
    def _():
      o_ref[...] = o_ref[...] * (1.0 / N) + b_ref[...]

  return pl.pallas_call(
      body,
      grid=(GRID,),
      in_specs=[
          pl.BlockSpec((NC, BM, D), lambda i: (0, i, 0)),
          pl.BlockSpec((BM, D), lambda i: (i, 0)),
          pl.BlockSpec((BM, D), lambda i: (i, 0)),
          pl.BlockSpec((1, D), lambda i: (0, 0)),
      ],
      out_specs=pl.BlockSpec((1, D), lambda i: (0, 0)),
      out_shape=jax.ShapeDtypeStruct((1, D), jnp.float32),
  )(P, g, dinv_b, bvec)


@jax.jit
def _run(x, edge_index, W1, b1, W2, b2, W3, b3, W4, b4,
         gamma1, beta1, gamma2, beta2, gamma3, beta3):
  src = edge_index[0]
  dst = edge_index[1]
  pad_e = jnp.full((E_PAD - E,), N, dtype=jnp.int32)
  srcp = jnp.concatenate([src, pad_e])
  dstp = jnp.concatenate([dst, pad_e])
  x_pad = jnp.pad(x, ((0, N_PAD - N), (0, 0)))

  ones_t = jnp.ones((N_PAD, D), jnp.float32)
  zeros_t = jnp.zeros((ROWS_PER_TILE, D), jnp.float32)

  b1r = b1.reshape(1, D)
  b2r = b2.reshape(1, D)
  b3r = b3.reshape(1, D)
  b4r = b4.reshape(1, D)
  g1r = gamma1.reshape(1, D)
  g2r = gamma2.reshape(1, D)
  g3r = gamma3.reshape(1, D)
  be1r = beta1.reshape(1, D)
  be2r = beta2.reshape(1, D)
  be3r = beta3.reshape(1, D)

  deg_p = _propagate(ones_t, srcp, dstp, zeros_t)
  dinv_b = _prep_dinv(deg_p)

  g1 = _matmul_scale(x_pad, W1, dinv_b)
  P1 = _propagate(g1, srcp, dstp, zeros_t)
  g2 = _combine_matmul(P1, g1, dinv_b, b1r, g1r, be1r, W2)
  P2 = _propagate(g2, srcp, dstp, zeros_t)
  g3 = _combine_matmul(P2, g2, dinv_b, b2r, g2r, be2r, W3)
  P3 = _propagate(g3, srcp, dstp, zeros_t)
  g4 = _combine_matmul(P3, g3, dinv_b, b3r, g3r, be3r, W4)
  P4 = _propagate(g4, srcp, dstp, zeros_t)
  return _final_mean(P4, g4, dinv_b, b4r)


def kernel(x, edge_index, W1, b1, W2, b2, W3, b3, W4, b4,
           gamma1, beta1, gamma2, beta2, gamma3, beta3):
  return _run(x, edge_index, W1, b1, W2, b2, W3, b3, W4, b4,
              gamma1, beta1, gamma2, beta2, gamma3, beta3)
